# baseline (speedup 1.0000x reference)
import jax
import jax.numpy as jnp
from jax import lax
from jax.experimental import pallas as pl
from jax.experimental.pallas import tpu as pltpu
from jax.experimental.pallas import tpu_sc as plsc

N = 10000
E = 320000
H = 128
G = 512
HH = H // 2

NC = 2
NS = 16
EDGES_PER_TILE = E // NS
CHUNK = 80
NCHUNK = EDGES_PER_TILE // CHUNK
ROWS_MOST = 640
ROWS_LAST = N - ROWS_MOST * (NS - 1)
PACK = 16384
NBUF = 3


def _rsqrt16(s):
    i = plsc.bitcast(s, jnp.int32)
    i = jnp.int32(0x5F3759DF) - (i >> 1)
    y = plsc.bitcast(i, jnp.float32)
    for _ in range(3):
        y = y * (1.5 - 0.5 * s * y * y)
    return y


def _sc_edge_kernel(x2_hbm, pidx_hbm, px_hbm, py_hbm, pz_hbm, out_hbm,
                    px_v, py_v, pz_v, row_all, col_all,
                    w_v, msg_a, msg_b, msg_c, msg_d, agg_sh,
                    gsem_a, gsem_b, gsem_c, gsem_d):
    c = lax.axis_index("c")
    s = lax.axis_index("s")

    pltpu.sync_copy(px_hbm, px_v)
    pltpu.sync_copy(py_hbm, py_v)
    pltpu.sync_copy(pz_hbm, pz_v)
    pltpu.sync_copy(pidx_hbm.at[s], col_all)

    coff = c * N

    def _uchunk(ci, _):
        def _ugrp(j, _):
            sl = pl.ds(j * 16, 16)
            p16 = col_all[ci, sl]
            row_all[ci, sl] = p16 >> 14
            col_all[ci, sl] = (p16 & (PACK - 1)) + coff
            return 0
        lax.fori_loop(0, CHUNK // 16, _ugrp, 0)
        return 0
    lax.fori_loop(0, NCHUNK, _uchunk, 0)

    def _zrow(i, _):
        for h in range(HH // 16):
            msg_a[i, pl.ds(h * 16, 16)] = jnp.zeros((16,), jnp.float32)
        return 0
    lax.fori_loop(0, CHUNK, _zrow, 0)
    row0 = pl.multiple_of(s * ROWS_MOST, ROWS_MOST)
    nrows = jnp.where(s == NS - 1, ROWS_LAST, ROWS_MOST)

    def _zcopy(z, _):
        off = pl.multiple_of(row0 + z * CHUNK, CHUNK)
        pltpu.sync_copy(msg_a, agg_sh.at[pl.ds(off, CHUNK)])
        return 0
    lax.fori_loop(0, nrows // CHUNK, _zcopy, 0)
    plsc.subcore_barrier()

    msgs = (msg_a, msg_b, msg_c, msg_d)
    gsems = (gsem_a, gsem_b, gsem_c, gsem_d)

    def _wcomp(ci):
        def _wgrp(j, _):
            sl = pl.ds(j * 16, 16)
            r16 = row_all[ci, sl]
            c16 = col_all[ci, sl] - coff
            dx = plsc.load_gather(px_v, [r16]) - plsc.load_gather(px_v, [c16])
            dy = plsc.load_gather(py_v, [r16]) - plsc.load_gather(py_v, [c16])
            dz = plsc.load_gather(pz_v, [r16]) - plsc.load_gather(pz_v, [c16])
            sq = dx * dx + dy * dy + dz * dz + 1e-12
            d = sq * _rsqrt16(sq)
            w_v[pl.ds(j * 16, 16)] = 1.0 / (1.0 + d)
            return 0
        lax.fori_loop(0, CHUNK // 16, _wgrp, 0)

    def _scale(msg_v):
        def _sgrp(j, _):
            wvec = w_v[pl.ds(j * 16, 16)]
            for l in range(16):
                e = j * 16 + l
                ws = wvec[l]
                for h in range(HH // 16):
                    sl = pl.ds(h * 16, 16)
                    msg_v[e, sl] = msg_v[e, sl] * ws
            return 0
        lax.fori_loop(0, CHUNK // 16, _sgrp, 0)

    def _quad(ci0, nb):
        gats = []
        for b in range(nb):
            gats.append(pltpu.async_copy(
                x2_hbm.at[col_all.at[ci0 + b]], msgs[b], gsems[b]))
        for b in range(nb):
            _wcomp(ci0 + b)
            gats[b].wait()
            _scale(msgs[b])
            pltpu.sync_copy(msgs[b], agg_sh.at[row_all.at[ci0 + b]], add=True)

    def _loop(p, _):
        _quad(p * 4, 4)
        return 0
    lax.fori_loop(0, NCHUNK // 4, _loop, 0)
    _quad(NCHUNK - 2, 2)

    plsc.subcore_barrier()

    @pl.when(s < NS - 1)
    def _wb_most():
        pltpu.sync_copy(agg_sh.at[pl.ds(row0, ROWS_MOST)],
                        out_hbm.at[c, pl.ds(row0, ROWS_MOST)])

    @pl.when(s == NS - 1)
    def _wb_last():
        pltpu.sync_copy(agg_sh.at[pl.ds(row0, ROWS_LAST)],
                        out_hbm.at[c, pl.ds(row0, ROWS_LAST)])


def _sc_edge(x2, pidx, px, py, pz):
    mesh = plsc.VectorSubcoreMesh(core_axis_name="c", subcore_axis_name="s")
    f = pl.kernel(
        _sc_edge_kernel, mesh=mesh,
        out_type=jax.ShapeDtypeStruct((NC, N, HH), jnp.float32),
        scratch_types=[
            pltpu.VMEM((N,), jnp.float32),
            pltpu.VMEM((N,), jnp.float32),
            pltpu.VMEM((N,), jnp.float32),
            pltpu.VMEM((NCHUNK, CHUNK), jnp.int32),
            pltpu.VMEM((NCHUNK, CHUNK), jnp.int32),
            pltpu.VMEM((CHUNK,), jnp.float32),
            pltpu.VMEM((CHUNK, HH), jnp.float32),
            pltpu.VMEM((CHUNK, HH), jnp.float32),
            pltpu.VMEM((CHUNK, HH), jnp.float32),
            pltpu.VMEM((CHUNK, HH), jnp.float32),
            pltpu.VMEM_SHARED((N, HH), jnp.float32),
            pltpu.SemaphoreType.DMA,
            pltpu.SemaphoreType.DMA,
            pltpu.SemaphoreType.DMA,
            pltpu.SemaphoreType.DMA,
        ],
        compiler_params=pltpu.CompilerParams(needs_layout_passes=False, use_tc_tiling_on_sc=False),
    )
    return f(x2, pidx, px, py, pz)


BLK = 1000
NB = N // BLK


def _tc_dense_kernel(x_ref, aggl_ref, aggr_ref, n2g_ref, wm_ref, w1_ref, b1_ref,
                     w2_ref, b2_ref, w3_ref, b3_ref, w4_ref, b4_ref,
                     out_ref, xg_acc):
    i = pl.program_id(0)
    agg = jnp.concatenate([aggl_ref[0], aggr_ref[0]], axis=-1)
    a = x_ref[...] + agg
    xl = a @ wm_ref[...]
    xl = xl * jax.nn.sigmoid(xl)
    xl = xl @ w1_ref[...] + b1_ref[...]
    xl = xl * jax.nn.sigmoid(xl)
    xl = xl @ w2_ref[...] + b2_ref[...]

    ids = n2g_ref[0, 0, :]
    gidx = lax.broadcasted_iota(jnp.int32, (G, BLK), 0)
    oh = (gidx == ids[None, :]).astype(jnp.float32)
    part = jax.lax.dot(oh, xl, preferred_element_type=jnp.float32)

    @pl.when(i == 0)
    def _init():
        xg_acc[...] = part

    @pl.when(i > 0)
    def _acc():
        xg_acc[...] = xg_acc[...] + part

    @pl.when(i == NB - 1)
    def _head():
        xg = xg_acc[...]
        h1 = xg @ w3_ref[...] + b3_ref[...]
        h1 = h1 * jax.nn.sigmoid(h1)
        out_ref[...] = h1 @ w4_ref[...] + b4_ref[...]


def _tc_dense(x, agg2, n2g_r, W_msg, W1, b1, W2, b2, W3, b3, W4p, b4p):
    full = lambda shape: pl.BlockSpec(shape, lambda i: tuple(0 for _ in shape))
    return pl.pallas_call(
        _tc_dense_kernel,
        grid=(NB,),
        in_specs=[
            pl.BlockSpec((BLK, H), lambda i: (i, 0)),
            pl.BlockSpec((1, BLK, HH), lambda i: (0, i, 0)),
            pl.BlockSpec((1, BLK, HH), lambda i: (1, i, 0)),
            pl.BlockSpec((1, 1, BLK), lambda i: (i, 0, 0)),
            full((H, H)),
            full((H, H)), full((1, H)),
            full((H, H)), full((1, H)),
            full((H, H)), full((1, H)),
            full((H, H)), full((1, H)),
        ],
        out_specs=pl.BlockSpec((G, H), lambda i: (0, 0)),
        out_shape=jax.ShapeDtypeStruct((G, H), jnp.float32),
        scratch_shapes=[pltpu.VMEM((G, H), jnp.float32)],
    )(x, agg2, agg2, n2g_r, W_msg, W1, b1, W2, b2, W3, b3, W4p, b4p)


def kernel(x, pos, edge_index, node2graph, W_msg, W1, b1, W2, b2, W3, b3, W4, b4):
    pidx = (edge_index[0] * PACK + edge_index[1]).reshape(NS, NCHUNK, CHUNK)
    x2 = jnp.concatenate([x[:, :HH], x[:, HH:]], axis=0)
    px = pos[:, 0]
    py = pos[:, 1]
    pz = pos[:, 2]
    agg2 = _sc_edge(x2, pidx, px, py, pz)

    n2g_r = node2graph.reshape(NB, 1, BLK)
    W4p = jnp.pad(W4, ((0, 0), (0, H - 1)))
    b4p = jnp.pad(b4, (0, H - 1)).reshape(1, H)
    e_full = _tc_dense(x, agg2, n2g_r, W_msg, W1, b1.reshape(1, H),
                       W2, b2.reshape(1, H), W3, b3.reshape(1, H), W4p, b4p)
    return e_full[:, :1]


# speedup vs baseline: 1.0863x; 1.0863x over previous
import jax
import jax.numpy as jnp
from jax import lax
from jax.experimental import pallas as pl
from jax.experimental.pallas import tpu as pltpu
from jax.experimental.pallas import tpu_sc as plsc

N = 10000
E = 320000
H = 128
G = 512
HH = H // 2

NC = 2
NS = 16
EDGES_PER_TILE = E // NS
CHUNK = 80
NCHUNK = EDGES_PER_TILE // CHUNK
ROWS_MOST = 640
ROWS_LAST = N - ROWS_MOST * (NS - 1)
PACK = 16384
NBUF = 3


def _rsqrt16(s):
    i = plsc.bitcast(s, jnp.int32)
    i = jnp.int32(0x5F3759DF) - (i >> 1)
    y = plsc.bitcast(i, jnp.float32)
    for _ in range(3):
        y = y * (1.5 - 0.5 * s * y * y)
    return y


def _sc_edge_kernel(x2_hbm, pidx_hbm, px_hbm, py_hbm, pz_hbm, out_hbm,
                    px_v, py_v, pz_v, row_all, col_all,
                    w_v, msg_a, msg_b, msg_c, msg_d, agg_sh,
                    gsem_a, gsem_b, gsem_c, gsem_d):
    c = lax.axis_index("c")
    s = lax.axis_index("s")

    pltpu.sync_copy(px_hbm, px_v)
    pltpu.sync_copy(py_hbm, py_v)
    pltpu.sync_copy(pz_hbm, pz_v)
    pltpu.sync_copy(pidx_hbm.at[s], col_all)

    coff = c * N

    def _uchunk(ci, _):
        def _ugrp(j, _):
            sl = pl.ds(j * 16, 16)
            p16 = col_all[ci, sl]
            row_all[ci, sl] = p16 >> 14
            col_all[ci, sl] = (p16 & (PACK - 1)) + coff
            return 0
        lax.fori_loop(0, CHUNK // 16, _ugrp, 0)
        return 0
    lax.fori_loop(0, NCHUNK, _uchunk, 0)

    def _zrow(i, _):
        for h in range(HH // 16):
            msg_a[i, pl.ds(h * 16, 16)] = jnp.zeros((16,), jnp.float32)
        return 0
    lax.fori_loop(0, CHUNK, _zrow, 0)
    row0 = pl.multiple_of(s * ROWS_MOST, ROWS_MOST)
    nrows = jnp.where(s == NS - 1, ROWS_LAST, ROWS_MOST)

    def _zcopy(z, _):
        off = pl.multiple_of(row0 + z * CHUNK, CHUNK)
        pltpu.sync_copy(msg_a, agg_sh.at[pl.ds(off, CHUNK)])
        return 0
    lax.fori_loop(0, nrows // CHUNK, _zcopy, 0)
    plsc.subcore_barrier()

    msgs = (msg_a, msg_b, msg_c, msg_d)
    gsems = (gsem_a, gsem_b, gsem_c, gsem_d)

    def _wcomp(ci, b):
        def _wgrp(j, _):
            sl = pl.ds(j * 16, 16)
            r16 = row_all[ci, sl]
            c16 = col_all[ci, sl] - coff
            dx = plsc.load_gather(px_v, [r16]) - plsc.load_gather(px_v, [c16])
            dy = plsc.load_gather(py_v, [r16]) - plsc.load_gather(py_v, [c16])
            dz = plsc.load_gather(pz_v, [r16]) - plsc.load_gather(pz_v, [c16])
            sq = dx * dx + dy * dy + dz * dz + 1e-12
            d = sq * _rsqrt16(sq)
            w_v[b, pl.ds(j * 16, 16)] = 1.0 / (1.0 + d)
            return 0
        lax.fori_loop(0, CHUNK // 16, _wgrp, 0)

    def _scale(msg_v, b):
        def _sgrp(j, _):
            wvec = w_v[b, pl.ds(j * 16, 16)]
            for l in range(16):
                e = j * 16 + l
                ws = wvec[l]
                for h in range(HH // 16):
                    sl = pl.ds(h * 16, 16)
                    msg_v[e, sl] = msg_v[e, sl] * ws
            return 0
        lax.fori_loop(0, CHUNK // 16, _sgrp, 0)

    def _quad(ci0, nb):
        gats = []
        for b in range(nb):
            gats.append(pltpu.async_copy(
                x2_hbm.at[col_all.at[ci0 + b]], msgs[b], gsems[b]))
        for b in range(nb):
            _wcomp(ci0 + b, b)
        for b in range(nb):
            gats[b].wait()
        for b in range(nb):
            _scale(msgs[b], b)
        scats = []
        for b in range(nb):
            scats.append(pltpu.async_copy(
                msgs[b], agg_sh.at[row_all.at[ci0 + b]], gsems[b], add=True))
        for b in range(nb):
            scats[b].wait()

    def _loop(p, _):
        _quad(p * 4, 4)
        return 0
    lax.fori_loop(0, NCHUNK // 4, _loop, 0)
    _quad(NCHUNK - 2, 2)

    plsc.subcore_barrier()

    @pl.when(s < NS - 1)
    def _wb_most():
        pltpu.sync_copy(agg_sh.at[pl.ds(row0, ROWS_MOST)],
                        out_hbm.at[c, pl.ds(row0, ROWS_MOST)])

    @pl.when(s == NS - 1)
    def _wb_last():
        pltpu.sync_copy(agg_sh.at[pl.ds(row0, ROWS_LAST)],
                        out_hbm.at[c, pl.ds(row0, ROWS_LAST)])


def _sc_edge(x2, pidx, px, py, pz):
    mesh = plsc.VectorSubcoreMesh(core_axis_name="c", subcore_axis_name="s")
    f = pl.kernel(
        _sc_edge_kernel, mesh=mesh,
        out_type=jax.ShapeDtypeStruct((NC, N, HH), jnp.float32),
        scratch_types=[
            pltpu.VMEM((N,), jnp.float32),
            pltpu.VMEM((N,), jnp.float32),
            pltpu.VMEM((N,), jnp.float32),
            pltpu.VMEM((NCHUNK, CHUNK), jnp.int32),
            pltpu.VMEM((NCHUNK, CHUNK), jnp.int32),
            pltpu.VMEM((4, CHUNK), jnp.float32),
            pltpu.VMEM((CHUNK, HH), jnp.float32),
            pltpu.VMEM((CHUNK, HH), jnp.float32),
            pltpu.VMEM((CHUNK, HH), jnp.float32),
            pltpu.VMEM((CHUNK, HH), jnp.float32),
            pltpu.VMEM_SHARED((N, HH), jnp.float32),
            pltpu.SemaphoreType.DMA,
            pltpu.SemaphoreType.DMA,
            pltpu.SemaphoreType.DMA,
            pltpu.SemaphoreType.DMA,
        ],
        compiler_params=pltpu.CompilerParams(needs_layout_passes=False, use_tc_tiling_on_sc=False),
    )
    return f(x2, pidx, px, py, pz)


BLK = 1000
NB = N // BLK


def _tc_dense_kernel(x_ref, aggl_ref, aggr_ref, n2g_ref, wm_ref, w1_ref, b1_ref,
                     w2_ref, b2_ref, w3_ref, b3_ref, w4_ref, b4_ref,
                     out_ref, xg_acc):
    i = pl.program_id(0)
    agg = jnp.concatenate([aggl_ref[0], aggr_ref[0]], axis=-1)
    a = x_ref[...] + agg
    xl = a @ wm_ref[...]
    xl = xl * jax.nn.sigmoid(xl)
    xl = xl @ w1_ref[...] + b1_ref[...]
    xl = xl * jax.nn.sigmoid(xl)
    xl = xl @ w2_ref[...] + b2_ref[...]

    ids = n2g_ref[0, 0, :]
    gidx = lax.broadcasted_iota(jnp.int32, (G, BLK), 0)
    oh = (gidx == ids[None, :]).astype(jnp.float32)
    part = jax.lax.dot(oh, xl, preferred_element_type=jnp.float32)

    @pl.when(i == 0)
    def _init():
        xg_acc[...] = part

    @pl.when(i > 0)
    def _acc():
        xg_acc[...] = xg_acc[...] + part

    @pl.when(i == NB - 1)
    def _head():
        xg = xg_acc[...]
        h1 = xg @ w3_ref[...] + b3_ref[...]
        h1 = h1 * jax.nn.sigmoid(h1)
        out_ref[...] = h1 @ w4_ref[...] + b4_ref[...]


def _tc_dense(x, agg2, n2g_r, W_msg, W1, b1, W2, b2, W3, b3, W4p, b4p):
    full = lambda shape: pl.BlockSpec(shape, lambda i: tuple(0 for _ in shape))
    return pl.pallas_call(
        _tc_dense_kernel,
        grid=(NB,),
        in_specs=[
            pl.BlockSpec((BLK, H), lambda i: (i, 0)),
            pl.BlockSpec((1, BLK, HH), lambda i: (0, i, 0)),
            pl.BlockSpec((1, BLK, HH), lambda i: (1, i, 0)),
            pl.BlockSpec((1, 1, BLK), lambda i: (i, 0, 0)),
            full((H, H)),
            full((H, H)), full((1, H)),
            full((H, H)), full((1, H)),
            full((H, H)), full((1, H)),
            full((H, H)), full((1, H)),
        ],
        out_specs=pl.BlockSpec((G, H), lambda i: (0, 0)),
        out_shape=jax.ShapeDtypeStruct((G, H), jnp.float32),
        scratch_shapes=[pltpu.VMEM((G, H), jnp.float32)],
    )(x, agg2, agg2, n2g_r, W_msg, W1, b1, W2, b2, W3, b3, W4p, b4p)


def kernel(x, pos, edge_index, node2graph, W_msg, W1, b1, W2, b2, W3, b3, W4, b4):
    pidx = (edge_index[0] * PACK + edge_index[1]).reshape(NS, NCHUNK, CHUNK)
    x2 = jnp.concatenate([x[:, :HH], x[:, HH:]], axis=0)
    px = pos[:, 0]
    py = pos[:, 1]
    pz = pos[:, 2]
    agg2 = _sc_edge(x2, pidx, px, py, pz)

    n2g_r = node2graph.reshape(NB, 1, BLK)
    W4p = jnp.pad(W4, ((0, 0), (0, H - 1)))
    b4p = jnp.pad(b4, (0, H - 1)).reshape(1, H)
    e_full = _tc_dense(x, agg2, n2g_r, W_msg, W1, b1.reshape(1, H),
                       W2, b2.reshape(1, H), W3, b3.reshape(1, H), W4p, b4p)
    return e_full[:, :1]


# speedup vs baseline: 2.1945x; 2.0201x over previous
import jax
import jax.numpy as jnp
from jax import lax
from jax.experimental import pallas as pl
from jax.experimental.pallas import tpu as pltpu
from jax.experimental.pallas import tpu_sc as plsc

N = 10000
E = 320000
H = 128
G = 512

NC = 2
NS = 16
NW = NC * NS
EDGES_PER_TILE = E // NW
CHUNK = 80
NCHUNK = EDGES_PER_TILE // CHUNK
ROWS_MOST = 640
ROWS_LAST = N - ROWS_MOST * (NS - 1)
PACK = 16384
SLAB = 25


def _rsqrt16(s):
    i = plsc.bitcast(s, jnp.int32)
    i = jnp.int32(0x5F3759DF) - (i >> 1)
    y = plsc.bitcast(i, jnp.float32)
    for _ in range(3):
        y = y * (1.5 - 0.5 * s * y * y)
    return y


def _sc_edge_kernel(x_hbm, pidx_hbm, pxy_hbm, pz_hbm, out_hbm,
                    pxy_v, pz_v, pidx_all,
                    row_a, row_b, col_a, col_b, w_a, w_b, msg_a, msg_b,
                    agg_sh, sem_a, sem_b):
    c = lax.axis_index("c")
    s = lax.axis_index("s")
    wid = s * NC + c

    pltpu.sync_copy(pxy_hbm, pxy_v)
    pltpu.sync_copy(pz_hbm, pz_v)

    def _zrow(i, _):
        for h in range(H // 16):
            msg_a[i, pl.ds(h * 16, 16)] = jnp.zeros((16,), jnp.float32)
        return 0
    lax.fori_loop(0, CHUNK, _zrow, 0)
    row0 = pl.multiple_of(s * ROWS_MOST, ROWS_MOST)
    nrows = jnp.where(s == NS - 1, ROWS_LAST, ROWS_MOST)

    def _zcopy(z, _):
        off = pl.multiple_of(row0 + z * CHUNK, CHUNK)
        pltpu.sync_copy(msg_a, agg_sh.at[pl.ds(off, CHUNK)])
        return 0
    lax.fori_loop(0, nrows // CHUNK, _zcopy, 0)
    plsc.subcore_barrier()

    bufs = ((row_a, col_a, w_a, msg_a, sem_a),
            (row_b, col_b, w_b, msg_b, sem_b))

    def _unpack(ci, row_v, col_v):
        def _ugrp(j, _):
            sl = pl.ds(j * 16, 16)
            p16 = pidx_all[pl.ds(ci * CHUNK + j * 16, 16)]
            row_v[sl] = p16 >> 14
            col_v[sl] = p16 & (PACK - 1)
            return 0
        lax.fori_loop(0, CHUNK // 16, _ugrp, 0)

    def _wcomp(row_v, col_v, w_v):
        def _wgrp(j, _):
            sl = pl.ds(j * 16, 16)
            r16 = row_v[sl]
            c16 = col_v[sl]
            gr = plsc.load_gather(pxy_v, [r16])
            gc = plsc.load_gather(pxy_v, [c16])
            dx = (plsc.bitcast(gr & jnp.int32(-65536), jnp.float32)
                  - plsc.bitcast(gc & jnp.int32(-65536), jnp.float32))
            dy = (plsc.bitcast(gr << 16, jnp.float32)
                  - plsc.bitcast(gc << 16, jnp.float32))
            dz = plsc.load_gather(pz_v, [r16]) - plsc.load_gather(pz_v, [c16])
            sq = dx * dx + dy * dy + dz * dz + 1e-12
            d = sq * _rsqrt16(sq)
            w_v[sl] = 1.0 / (1.0 + d)
            return 0
        lax.fori_loop(0, CHUNK // 16, _wgrp, 0)

    def _scale(msg_v, w_v):
        def _sgrp(j, _):
            wvec = w_v[pl.ds(j * 16, 16)]
            for l in range(16):
                e = j * 16 + l
                ws = wvec[l]
                for h in range(H // 16):
                    sl = pl.ds(h * 16, 16)
                    msg_v[e, sl] = msg_v[e, sl] * ws
            return 0
        lax.fori_loop(0, CHUNK // 16, _sgrp, 0)

    def _group(ci0, nb):
        for b in range(nb):
            row_v, col_v, _, _, _ = bufs[b]
            _unpack(ci0 + b, row_v, col_v)
        gats = []
        for b in range(nb):
            _, col_v, _, msg_v, sem = bufs[b]
            gats.append(pltpu.async_copy(x_hbm.at[col_v], msg_v, sem))
        for b in range(nb):
            row_v, col_v, w_v, _, _ = bufs[b]
            _wcomp(row_v, col_v, w_v)
        for b in range(nb):
            gats[b].wait()
        for b in range(nb):
            _, _, w_v, msg_v, _ = bufs[b]
            _scale(msg_v, w_v)
        scats = []
        for b in range(nb):
            row_v, _, _, msg_v, sem = bufs[b]
            scats.append(pltpu.async_copy(msg_v, agg_sh.at[row_v], sem,
                                          add=True))
        for b in range(nb):
            scats[b].wait()

    def _pair(p, _):
        _group(2 * p, 2)
        return 0

    for s5 in range(NCHUNK // SLAB):
        off = pl.multiple_of(wid * EDGES_PER_TILE + s5 * SLAB * CHUNK,
                             SLAB * CHUNK)
        pltpu.sync_copy(pidx_hbm.at[pl.ds(off, SLAB * CHUNK)], pidx_all)
        lax.fori_loop(0, SLAB // 2, _pair, 0)
        _group(SLAB - 1, 1)

    plsc.subcore_barrier()

    @pl.when(s < NS - 1)
    def _wb_most():
        pltpu.sync_copy(agg_sh.at[pl.ds(row0, ROWS_MOST)],
                        out_hbm.at[c, pl.ds(row0, ROWS_MOST)])

    @pl.when(s == NS - 1)
    def _wb_last():
        pltpu.sync_copy(agg_sh.at[pl.ds(row0, ROWS_LAST)],
                        out_hbm.at[c, pl.ds(row0, ROWS_LAST)])


def _sc_edge(x, pidx, pxy, pz):
    mesh = plsc.VectorSubcoreMesh(core_axis_name="c", subcore_axis_name="s")
    f = pl.kernel(
        _sc_edge_kernel, mesh=mesh,
        out_type=jax.ShapeDtypeStruct((NC, N, H), jnp.float32),
        scratch_types=[
            pltpu.VMEM((N,), jnp.int32),
            pltpu.VMEM((N,), jnp.float32),
            pltpu.VMEM((SLAB * CHUNK,), jnp.int32),
            pltpu.VMEM((CHUNK,), jnp.int32),
            pltpu.VMEM((CHUNK,), jnp.int32),
            pltpu.VMEM((CHUNK,), jnp.int32),
            pltpu.VMEM((CHUNK,), jnp.int32),
            pltpu.VMEM((CHUNK,), jnp.float32),
            pltpu.VMEM((CHUNK,), jnp.float32),
            pltpu.VMEM((CHUNK, H), jnp.float32),
            pltpu.VMEM((CHUNK, H), jnp.float32),
            pltpu.VMEM_SHARED((N, H), jnp.float32),
            pltpu.SemaphoreType.DMA,
            pltpu.SemaphoreType.DMA,
        ],
        compiler_params=pltpu.CompilerParams(needs_layout_passes=False),
    )
    return f(x, pidx, pxy, pz)


BLK = 1000
NB = N // BLK


def _tc_dense_kernel(x_ref, agg_ref, n2g_ref, wm_ref, w1_ref, b1_ref,
                     w2_ref, b2_ref, w3_ref, b3_ref, w4_ref, b4_ref,
                     out_ref, xg_acc):
    i = pl.program_id(0)
    a = x_ref[...] + agg_ref[0] + agg_ref[1]
    xl = a @ wm_ref[...]
    xl = xl * jax.nn.sigmoid(xl)
    xl = xl @ w1_ref[...] + b1_ref[...]
    xl = xl * jax.nn.sigmoid(xl)
    xl = xl @ w2_ref[...] + b2_ref[...]

    ids = n2g_ref[0, 0, :]
    gidx = lax.broadcasted_iota(jnp.int32, (G, BLK), 0)
    oh = (gidx == ids[None, :]).astype(jnp.float32)
    part = jax.lax.dot(oh, xl, preferred_element_type=jnp.float32)

    @pl.when(i == 0)
    def _init():
        xg_acc[...] = part

    @pl.when(i > 0)
    def _acc():
        xg_acc[...] = xg_acc[...] + part

    @pl.when(i == NB - 1)
    def _head():
        xg = xg_acc[...]
        h1 = xg @ w3_ref[...] + b3_ref[...]
        h1 = h1 * jax.nn.sigmoid(h1)
        out_ref[...] = h1 @ w4_ref[...] + b4_ref[...]


def _tc_dense(x, agg2, n2g_r, W_msg, W1, b1, W2, b2, W3, b3, W4p, b4p):
    full = lambda shape: pl.BlockSpec(shape, lambda i: tuple(0 for _ in shape))
    return pl.pallas_call(
        _tc_dense_kernel,
        grid=(NB,),
        in_specs=[
            pl.BlockSpec((BLK, H), lambda i: (i, 0)),
            pl.BlockSpec((NC, BLK, H), lambda i: (0, i, 0)),
            pl.BlockSpec((1, 1, BLK), lambda i: (i, 0, 0)),
            full((H, H)),
            full((H, H)), full((1, H)),
            full((H, H)), full((1, H)),
            full((H, H)), full((1, H)),
            full((H, H)), full((1, H)),
        ],
        out_specs=pl.BlockSpec((G, H), lambda i: (0, 0)),
        out_shape=jax.ShapeDtypeStruct((G, H), jnp.float32),
        scratch_shapes=[pltpu.VMEM((G, H), jnp.float32)],
    )(x, agg2, n2g_r, W_msg, W1, b1, W2, b2, W3, b3, W4p, b4p)


def kernel(x, pos, edge_index, node2graph, W_msg, W1, b1, W2, b2, W3, b3, W4, b4):
    pidx = edge_index[0] * PACK + edge_index[1]
    pxb = lax.bitcast_convert_type(pos[:, 0].astype(jnp.bfloat16), jnp.uint16)
    pyb = lax.bitcast_convert_type(pos[:, 1].astype(jnp.bfloat16), jnp.uint16)
    pxy = (pxb.astype(jnp.int32) << 16) | pyb.astype(jnp.int32)
    pz = pos[:, 2]
    agg2 = _sc_edge(x, pidx, pxy, pz)

    n2g_r = node2graph.reshape(NB, 1, BLK)
    W4p = jnp.pad(W4, ((0, 0), (0, H - 1)))
    b4p = jnp.pad(b4, (0, H - 1)).reshape(1, H)
    e_full = _tc_dense(x, agg2, n2g_r, W_msg, W1, b1.reshape(1, H),
                       W2, b2.reshape(1, H), W3, b3.reshape(1, H), W4p, b4p)
    return e_full[:, :1]
